# baseline (speedup 1.0000x reference)
import functools

import jax
import jax.numpy as jnp
from jax import lax
from jax.experimental import pallas as pl
from jax.experimental.pallas import tpu as pltpu
from jax.experimental.pallas import tpu_sc as plsc

N = 100000
E = 1600000
NP = 102400
EP = 1638400
EP8 = EP // 8
BN = 2048
BEP = 1600

NW = 32
CH = 128
EPH = EP // 2
EPH8 = EP8 // 2
NCHH = EPH // CH // NW
NCH0H = 144
NCH1H = 2 * NCHH - NCH0H
GG = 16
GGR = GG * CH
G = 8
NGH = NCHH // G
GR = G * CH
NZB = 320


def _sc_gather(tables):
    nt = len(tables)
    mesh = plsc.VectorSubcoreMesh(core_axis_name="c", subcore_axis_name="s")

    @functools.partial(
        pl.kernel,
        out_type=tuple(jax.ShapeDtypeStruct((EPH, 16), jnp.float32)
                       for _ in range(nt)),
        mesh=mesh,
        scratch_types=[pltpu.VMEM((GG, CH), jnp.int32) for _ in range(nt)]
                      + [pltpu.VMEM((GGR, 16), jnp.float32) for _ in range(nt)]
                      + [pltpu.SemaphoreType.DMA],
        compiler_params=pltpu.CompilerParams(use_tc_tiling_on_sc=False),
    )
    def k(*refs):
        tabs = refs[0:nt]
        idxs = refs[nt:2 * nt]
        outs = refs[2 * nt:3 * nt]
        ivs = refs[3 * nt:4 * nt]
        rvs = refs[4 * nt:5 * nt]
        sem = refs[5 * nt]
        c = lax.axis_index("c")
        s = lax.axis_index("s")
        crow0 = jnp.where(c == 0, s * NCH0H, 16 * NCH0H + s * NCH1H)
        ng = jnp.where(c == 0, NCH0H // GG, NCH1H // GG)

        def group(g, carry):
            gr0 = crow0 + g * GG
            for t in range(nt):
                pltpu.sync_copy(idxs[t].at[pl.ds(gr0, GG), :], ivs[t])
            descs = []
            for t in range(nt):
                for j in range(GG):
                    descs.append(pltpu.async_copy(
                        tabs[t].at[ivs[t].at[j]],
                        rvs[t].at[pl.ds(j * CH, CH), :], sem))
            for d in descs:
                d.wait()
            for t in range(nt):
                pltpu.sync_copy(
                    rvs[t], outs[t].at[pl.ds(gr0 * CH, GGR), :])
            return carry

        lax.fori_loop(0, ng, group, 0)

    args = [t for t, _ in tables] + [i for _, i in tables]
    res = k(*args)
    if nt == 1:
        res = (res,)
    return tuple(r.reshape(EPH8, 128) for r in res)


def _sc_scatter(eh_p, dst2):
    mesh = plsc.VectorSubcoreMesh(core_axis_name="c", subcore_axis_name="s")
    NPS = NP // 16

    @functools.partial(
        pl.kernel,
        out_type=jax.ShapeDtypeStruct((2, NP, 16), jnp.float32),
        mesh=mesh,
        scratch_types=[
            pltpu.VMEM((G, CH), jnp.int32),
            pltpu.VMEM((GR, 16), jnp.float32),
            pltpu.VMEM((NZB, 16), jnp.float32),
            pltpu.VMEM_SHARED((NP, 16), jnp.float32),
            pltpu.SemaphoreType.DMA,
        ],
        compiler_params=pltpu.CompilerParams(use_tc_tiling_on_sc=False),
    )
    def k(eh_hbm, dst_hbm, out_hbm, di_v, er_v, z_v, acc, sem):
        c = lax.axis_index("c")
        s = lax.axis_index("s")
        wid = s * 2 + c

        def zrow(i, carry):
            z_v[i, :] = jnp.zeros((16,), jnp.float32)
            return carry
        lax.fori_loop(0, NZB, zrow, 0)

        def zcp(i, carry):
            pltpu.sync_copy(z_v, acc.at[pl.ds(s * NPS + i * NZB, NZB), :])
            return carry
        lax.fori_loop(0, NPS // NZB, zcp, 0)
        plsc.subcore_barrier()

        crow0 = wid * NCHH

        def group(g, carry):
            gr0 = crow0 + g * G
            pltpu.sync_copy(dst_hbm.at[pl.ds(gr0, G), :], di_v)
            pltpu.sync_copy(eh_hbm.at[pl.ds(gr0 * CH, GR), :], er_v)
            descs = []
            for j in range(G):
                descs.append(pltpu.async_copy(
                    er_v.at[pl.ds(j * CH, CH), :],
                    acc.at[di_v.at[j]], sem, add=True))
            for d in descs:
                d.wait()
            return carry
        lax.fori_loop(0, NGH, group, 0)
        plsc.subcore_barrier()

        pltpu.sync_copy(acc.at[pl.ds(s * NPS, NPS), :],
                        out_hbm.at[c, pl.ds(s * NPS, NPS), :])

    return k(eh_p.reshape(EPH, 16), dst2)


def _tc_prep_body(nfeat_ref, flav_ref, emb_ref, nf_ref):
    fl = flav_ref[0, 0, :]
    oh = (fl[:, None] == lax.broadcasted_iota(jnp.int32, (BN, 16), 1))
    emb_rows = oh.astype(jnp.float32) @ emb_ref[...]
    nf_ref[:, 0:8] = nfeat_ref[...]
    nf_ref[:, 8:16] = emb_rows


def _tc_prep(nfeat, flav3, emb):
    grid = NP // BN
    return pl.pallas_call(
        _tc_prep_body,
        grid=(grid,),
        in_specs=[
            pl.BlockSpec((BN, 8), lambda i: (i, 0)),
            pl.BlockSpec((1, 1, BN), lambda i: (i, 0, 0)),
            pl.BlockSpec((16, 8), lambda i: (0, 0)),
        ],
        out_specs=pl.BlockSpec((BN, 16), lambda i: (i, 0)),
        out_shape=jax.ShapeDtypeStruct((NP, 16), jnp.float32),
    )(nfeat, flav3, emb)


def _tc_edge_body(nin, *refs):
    (g_refs, efp_ref, w_refs, wef_ref, b1_ref, w2_ref, b2_ref,
     out_ref) = (refs[0:nin], refs[nin], refs[nin + 1:2 * nin + 1],
                 refs[2 * nin + 1], refs[2 * nin + 2], refs[2 * nin + 3],
                 refs[2 * nin + 4], refs[2 * nin + 5])
    h = efp_ref[...] @ wef_ref[...] + b1_ref[...]
    for g_ref, w_ref in zip(g_refs, w_refs):
        h = h + g_ref[...] @ w_ref[...]
    h = jnp.maximum(h, 0.0)
    out_ref[...] = jnp.tanh(h @ w2_ref[...] + b2_ref[...])


def _tc_edge(gs, efp, ws, wef, b1t, w2bd, b2t):
    grid = EPH8 // BEP
    nin = len(gs)
    dh8 = wef.shape[1]
    return pl.pallas_call(
        functools.partial(_tc_edge_body, nin),
        grid=(grid,),
        in_specs=[pl.BlockSpec((BEP, 128), lambda i: (i, 0))
                  for _ in range(nin)]
                 + [pl.BlockSpec((BEP, 8), lambda i: (i, 0))]
                 + [pl.BlockSpec((128, dh8), lambda i: (0, 0))
                    for _ in range(nin)]
                 + [pl.BlockSpec((8, dh8), lambda i: (0, 0)),
                    pl.BlockSpec((1, dh8), lambda i: (0, 0)),
                    pl.BlockSpec((dh8, 128), lambda i: (0, 0)),
                    pl.BlockSpec((1, 128), lambda i: (0, 0))],
        out_specs=pl.BlockSpec((BEP, 128), lambda i: (i, 0)),
        out_shape=jax.ShapeDtypeStruct((EPH8, 128), jnp.float32),
    )(*gs, efp, *ws, wef, b1t, w2bd, b2t)


def _mlp_tanh_in(x, w1, b1, w2, b2):
    h = jnp.maximum(x @ w1 + b1, 0.0)
    return jnp.tanh(h @ w2 + b2)


def _tc_node0_body(nf_ref, ms0_ref, ms1_ref, ms2_ref, ms3_ref,
                   aw1, ab1, aw2, ab2, bw1, bb1, bw2, bb2, h_ref):
    nf = nf_ref[...]
    ms = (ms0_ref[...] + ms1_ref[...]) + (ms2_ref[...] + ms3_ref[...])
    r1 = _mlp_tanh_in(nf, aw1[...], ab1[...], aw2[...], ab2[...])
    r2 = _mlp_tanh_in(ms, bw1[...], bb1[...], bw2[...], bb2[...])
    h = jnp.concatenate([r1, r2], axis=1)
    h_ref[...] = h / jnp.sqrt(jnp.sum(h * h, axis=1, keepdims=True))


def _tc_node0(nf, mss, p):
    grid = NP // BN
    return pl.pallas_call(
        _tc_node0_body,
        grid=(grid,),
        in_specs=[pl.BlockSpec((BN, 16), lambda i: (i, 0))
                  for _ in range(5)]
                 + [pl.BlockSpec(w.shape, lambda i: (0, 0)) for w in
                    (p["a_w1"], p["a_b1"], p["a_w2"], p["a_b2"],
                     p["b_w1"], p["b_b1"], p["b_w2"], p["b_b2"])],
        out_specs=pl.BlockSpec((BN, 16), lambda i: (i, 0)),
        out_shape=jax.ShapeDtypeStruct((NP, 16), jnp.float32),
    )(nf, *mss, p["a_w1"], p["a_b1"], p["a_w2"], p["a_b2"],
      p["b_w1"], p["b_b1"], p["b_w2"], p["b_b2"])


def _tc_node1_body(nf_ref, h_ref, ms0_ref, ms1_ref, ms2_ref, ms3_ref,
                   aw1n, aw1h, ab1, aw2, ab2,
                   bw1, bb1, bw2, bb2,
                   jw1, jb1, jw2, jb2, jw3, jb3, out_ref):
    ms = (ms0_ref[...] + ms1_ref[...]) + (ms2_ref[...] + ms3_ref[...])
    t = nf_ref[...] @ aw1n[...] + h_ref[...] @ aw1h[...] + ab1[...]
    r1 = jnp.tanh(jnp.maximum(t, 0.0) @ aw2[...] + ab2[...])
    r2 = _mlp_tanh_in(ms, bw1[...], bb1[...], bw2[...], bb2[...])
    h = jnp.concatenate([r1, r2], axis=1)
    h = h / jnp.sqrt(jnp.sum(h * h, axis=1, keepdims=True))
    z = jnp.maximum(h @ jw1[...] + jb1[...], 0.0)
    z = jnp.maximum(z @ jw2[...] + jb2[...], 0.0)
    out_ref[...] = z @ jw3[...] + jb3[...]


def _tc_node1(nf, h, mss, p, pj):
    grid = NP // BN
    ws = (p["a_w1"][0:16], p["a_w1"][16:32], p["a_b1"], p["a_w2"], p["a_b2"],
          p["b_w1"], p["b_b1"], p["b_w2"], p["b_b2"],
          pj["w1"], pj["b1"], pj["w2"], pj["b2"], pj["w3"], pj["b3"])
    return pl.pallas_call(
        _tc_node1_body,
        grid=(grid,),
        in_specs=[pl.BlockSpec((BN, 16), lambda i: (i, 0))
                  for _ in range(6)]
                 + [pl.BlockSpec(w.shape, lambda i: (0, 0)) for w in ws],
        out_specs=pl.BlockSpec((BN, 3), lambda i: (i, 0)),
        out_shape=jax.ShapeDtypeStruct((NP, 3), jnp.float32),
    )(nf, h, *mss, *ws)


def _pad_rows(x, rows):
    return jnp.pad(x, ((0, rows - x.shape[0]),) + ((0, 0),) * (x.ndim - 1))


def _as2d(b):
    return b.reshape(1, -1)


def _bd(w):
    a, b = w.shape
    out = jnp.zeros((8 * a, 8 * b), w.dtype)
    for j in range(8):
        out = out.at[j * a:(j + 1) * a, j * b:(j + 1) * b].set(w)
    return out


def kernel(node_features, flavour_indices, edge_index, edge_feat, params):
    nfeat = _pad_rows(node_features.astype(jnp.float32), NP)
    flav = _pad_rows(flavour_indices.astype(jnp.int32), NP)
    flav3 = flav.reshape(NP // BN, 1, BN)
    src = edge_index[0].astype(jnp.int32)
    dst = edge_index[1].astype(jnp.int32)
    src = jnp.pad(src, (0, EP - E))
    dst = jnp.pad(dst, (0, EP - E), constant_values=NP - 1)
    NCR = EPH // CH
    src2 = src.reshape(EP // CH, CH)
    dst2 = dst.reshape(EP // CH, CH)
    src2h = [src2[0:NCR], src2[NCR:]]
    dst2h = [dst2[0:NCR], dst2[NCR:]]
    ef = jnp.pad(edge_feat.astype(jnp.float32), (0, EP - E))
    efp = ef.reshape(EP8, 8)
    efph = [efp[0:EPH8], efp[EPH8:]]

    p = params
    nf = _tc_prep(nfeat, flav3, p["emb"])

    e0 = p["e0"]
    w1 = e0["w1"]
    w0s = ([_bd(w1[0:16]), _bd(w1[16:32])], _bd(w1[32:33]),
           _as2d(jnp.tile(e0["b1"], 8)), _bd(e0["w2"]),
           _as2d(jnp.tile(e0["b2"], 8)))
    gnf, mps = [], []
    for hf in range(2):
        gdp, gsp = _sc_gather([(nf, dst2h[hf]), (nf, src2h[hf])])
        gnf.append((gdp, gsp))
        eh = _tc_edge([gdp, gsp], efph[hf], *w0s)
        mp = _sc_scatter(eh, dst2h[hf])
        mps += [mp[0], mp[1]]
    n0 = {k: (v if v.ndim == 2 else _as2d(v)) for k, v in p["n0"].items()}
    h = _tc_node0(nf, mps, n0)

    e1 = p["e1"]
    w1 = e1["w1"]
    w1s = ([_bd(w1[0:16]), _bd(w1[16:32]), _bd(w1[32:48]), _bd(w1[48:64])],
           _bd(w1[64:65]),
           _as2d(jnp.tile(e1["b1"], 8)), _bd(e1["w2"]),
           _as2d(jnp.tile(e1["b2"], 8)))
    mps = []
    for hf in range(2):
        ghd, ghs = _sc_gather([(h, dst2h[hf]), (h, src2h[hf])])
        gdp, gsp = gnf[hf]
        eh = _tc_edge([gdp, ghd, gsp, ghs], efph[hf], *w1s)
        mp = _sc_scatter(eh, dst2h[hf])
        mps += [mp[0], mp[1]]
    n1 = {k: (v if v.ndim == 2 else _as2d(v)) for k, v in p["n1"].items()}
    pj = {k: (v if v.ndim == 2 else _as2d(v)) for k, v in p["jet"].items()}
    out = _tc_node1(nf, h, mps, n1, pj)
    return out[:N]


# speedup vs baseline: 1.1606x; 1.1606x over previous
import functools

import jax
import jax.numpy as jnp
from jax import lax
from jax.experimental import pallas as pl
from jax.experimental.pallas import tpu as pltpu
from jax.experimental.pallas import tpu_sc as plsc

N = 100000
E = 1600000
NP = 102400
EP = 1638400
EP8 = EP // 8
BN = 2048
BEP = 1600

NW = 32
CH = 128
EPH = EP // 2
EPH8 = EP8 // 2
NCHH = EPH // CH // NW
NCH0H = 256
NCH1H = 2 * NCHH - NCH0H
GG = 16
GGR = GG * CH
G = 8
NGH = NCHH // G
GR = G * CH
NZB = 320


def _sc_gather(tables):
    nt = len(tables)
    mesh = plsc.VectorSubcoreMesh(core_axis_name="c", subcore_axis_name="s")

    @functools.partial(
        pl.kernel,
        out_type=tuple(jax.ShapeDtypeStruct((EPH, 16), jnp.float32)
                       for _ in range(nt)),
        mesh=mesh,
        scratch_types=[pltpu.VMEM((GG, CH), jnp.int32) for _ in range(nt)]
                      + [pltpu.VMEM((GGR, 16), jnp.float32) for _ in range(nt)]
                      + [pltpu.SemaphoreType.DMA],
        compiler_params=pltpu.CompilerParams(use_tc_tiling_on_sc=False),
    )
    def k(*refs):
        tabs = refs[0:nt]
        idxs = refs[nt:2 * nt]
        outs = refs[2 * nt:3 * nt]
        ivs = refs[3 * nt:4 * nt]
        rvs = refs[4 * nt:5 * nt]
        sem = refs[5 * nt]
        c = lax.axis_index("c")
        s = lax.axis_index("s")
        crow0 = jnp.where(c == 0, s * NCH0H, 16 * NCH0H + s * NCH1H)
        ng = jnp.where(c == 0, NCH0H // GG, NCH1H // GG)

        def group(g, carry):
            gr0 = crow0 + g * GG
            for t in range(nt):
                pltpu.sync_copy(idxs[t].at[pl.ds(gr0, GG), :], ivs[t])
            descs = []
            for t in range(nt):
                for j in range(GG):
                    descs.append(pltpu.async_copy(
                        tabs[t].at[ivs[t].at[j]],
                        rvs[t].at[pl.ds(j * CH, CH), :], sem))
            for d in descs:
                d.wait()
            for t in range(nt):
                pltpu.sync_copy(
                    rvs[t], outs[t].at[pl.ds(gr0 * CH, GGR), :])
            return carry

        lax.fori_loop(0, ng, group, 0)

    args = [t for t, _ in tables] + [i for _, i in tables]
    res = k(*args)
    if nt == 1:
        res = (res,)
    return tuple(r.reshape(EPH8, 128) for r in res)


def _sc_scatter(eh_p, dst2, init=None):
    mesh = plsc.VectorSubcoreMesh(core_axis_name="c", subcore_axis_name="s")
    NPS = NP // 16
    chained = init is not None

    @functools.partial(
        pl.kernel,
        out_type=jax.ShapeDtypeStruct((2, NP, 16), jnp.float32),
        mesh=mesh,
        scratch_types=[
            pltpu.VMEM((G, CH), jnp.int32),
            pltpu.VMEM((GR, 16), jnp.float32),
            pltpu.VMEM((NZB, 16), jnp.float32),
            pltpu.VMEM_SHARED((NP, 16), jnp.float32),
            pltpu.SemaphoreType.DMA,
        ],
        compiler_params=pltpu.CompilerParams(use_tc_tiling_on_sc=False),
    )
    def k(*refs):
        if chained:
            eh_hbm, dst_hbm, init_hbm, out_hbm = refs[0:4]
        else:
            eh_hbm, dst_hbm, out_hbm = refs[0:3]
            init_hbm = None
        di_v, er_v, z_v, acc, sem = refs[-5:]
        c = lax.axis_index("c")
        s = lax.axis_index("s")
        wid = s * 2 + c

        if chained:
            pltpu.sync_copy(init_hbm.at[c, pl.ds(s * NPS, NPS), :],
                            acc.at[pl.ds(s * NPS, NPS), :])
        else:
            def zrow(i, carry):
                z_v[i, :] = jnp.zeros((16,), jnp.float32)
                return carry
            lax.fori_loop(0, NZB, zrow, 0)

            def zcp(i, carry):
                pltpu.sync_copy(z_v,
                                acc.at[pl.ds(s * NPS + i * NZB, NZB), :])
                return carry
            lax.fori_loop(0, NPS // NZB, zcp, 0)
        plsc.subcore_barrier()

        crow0 = wid * NCHH

        def group(g, carry):
            gr0 = crow0 + g * G
            pltpu.sync_copy(dst_hbm.at[pl.ds(gr0, G), :], di_v)
            pltpu.sync_copy(eh_hbm.at[pl.ds(gr0 * CH, GR), :], er_v)
            descs = []
            for j in range(G):
                descs.append(pltpu.async_copy(
                    er_v.at[pl.ds(j * CH, CH), :],
                    acc.at[di_v.at[j]], sem, add=True))
            for d in descs:
                d.wait()
            return carry
        lax.fori_loop(0, NGH, group, 0)
        plsc.subcore_barrier()

        pltpu.sync_copy(acc.at[pl.ds(s * NPS, NPS), :],
                        out_hbm.at[c, pl.ds(s * NPS, NPS), :])

    if chained:
        return k(eh_p.reshape(EPH, 16), dst2, init)
    return k(eh_p.reshape(EPH, 16), dst2)


def _tc_prep_body(nfeat_ref, flav_ref, emb_ref, nf_ref):
    fl = flav_ref[0, 0, :]
    oh = (fl[:, None] == lax.broadcasted_iota(jnp.int32, (BN, 16), 1))
    emb_rows = oh.astype(jnp.float32) @ emb_ref[...]
    nf_ref[:, 0:8] = nfeat_ref[...]
    nf_ref[:, 8:16] = emb_rows


def _tc_prep(nfeat, flav3, emb):
    grid = NP // BN
    return pl.pallas_call(
        _tc_prep_body,
        grid=(grid,),
        in_specs=[
            pl.BlockSpec((BN, 8), lambda i: (i, 0)),
            pl.BlockSpec((1, 1, BN), lambda i: (i, 0, 0)),
            pl.BlockSpec((16, 8), lambda i: (0, 0)),
        ],
        out_specs=pl.BlockSpec((BN, 16), lambda i: (i, 0)),
        out_shape=jax.ShapeDtypeStruct((NP, 16), jnp.float32),
    )(nfeat, flav3, emb)


def _tc_edge_body(nin, *refs):
    (g_refs, efp_ref, w_refs, wef_ref, b1_ref, w2_ref, b2_ref,
     out_ref) = (refs[0:nin], refs[nin], refs[nin + 1:2 * nin + 1],
                 refs[2 * nin + 1], refs[2 * nin + 2], refs[2 * nin + 3],
                 refs[2 * nin + 4], refs[2 * nin + 5])
    h = efp_ref[...] @ wef_ref[...] + b1_ref[...]
    for g_ref, w_ref in zip(g_refs, w_refs):
        h = h + g_ref[...] @ w_ref[...]
    h = jnp.maximum(h, 0.0)
    out_ref[...] = jnp.tanh(h @ w2_ref[...] + b2_ref[...])


def _tc_edge(gs, efp, ws, wef, b1t, w2bd, b2t):
    grid = EPH8 // BEP
    nin = len(gs)
    dh8 = wef.shape[1]
    return pl.pallas_call(
        functools.partial(_tc_edge_body, nin),
        grid=(grid,),
        in_specs=[pl.BlockSpec((BEP, 128), lambda i: (i, 0))
                  for _ in range(nin)]
                 + [pl.BlockSpec((BEP, 8), lambda i: (i, 0))]
                 + [pl.BlockSpec((128, dh8), lambda i: (0, 0))
                    for _ in range(nin)]
                 + [pl.BlockSpec((8, dh8), lambda i: (0, 0)),
                    pl.BlockSpec((1, dh8), lambda i: (0, 0)),
                    pl.BlockSpec((dh8, 128), lambda i: (0, 0)),
                    pl.BlockSpec((1, 128), lambda i: (0, 0))],
        out_specs=pl.BlockSpec((BEP, 128), lambda i: (i, 0)),
        out_shape=jax.ShapeDtypeStruct((EPH8, 128), jnp.float32),
    )(*gs, efp, *ws, wef, b1t, w2bd, b2t)


def _mlp_tanh_in(x, w1, b1, w2, b2):
    h = jnp.maximum(x @ w1 + b1, 0.0)
    return jnp.tanh(h @ w2 + b2)


def _tc_node0_body(nf_ref, ms0_ref, ms1_ref,
                   aw1, ab1, aw2, ab2, bw1, bb1, bw2, bb2, h_ref):
    nf = nf_ref[...]
    ms = ms0_ref[...] + ms1_ref[...]
    r1 = _mlp_tanh_in(nf, aw1[...], ab1[...], aw2[...], ab2[...])
    r2 = _mlp_tanh_in(ms, bw1[...], bb1[...], bw2[...], bb2[...])
    h = jnp.concatenate([r1, r2], axis=1)
    h_ref[...] = h / jnp.sqrt(jnp.sum(h * h, axis=1, keepdims=True))


def _tc_node0(nf, mss, p):
    grid = NP // BN
    return pl.pallas_call(
        _tc_node0_body,
        grid=(grid,),
        in_specs=[pl.BlockSpec((BN, 16), lambda i: (i, 0))
                  for _ in range(3)]
                 + [pl.BlockSpec(w.shape, lambda i: (0, 0)) for w in
                    (p["a_w1"], p["a_b1"], p["a_w2"], p["a_b2"],
                     p["b_w1"], p["b_b1"], p["b_w2"], p["b_b2"])],
        out_specs=pl.BlockSpec((BN, 16), lambda i: (i, 0)),
        out_shape=jax.ShapeDtypeStruct((NP, 16), jnp.float32),
    )(nf, *mss, p["a_w1"], p["a_b1"], p["a_w2"], p["a_b2"],
      p["b_w1"], p["b_b1"], p["b_w2"], p["b_b2"])


def _tc_node1_body(nf_ref, h_ref, ms0_ref, ms1_ref,
                   aw1n, aw1h, ab1, aw2, ab2,
                   bw1, bb1, bw2, bb2,
                   jw1, jb1, jw2, jb2, jw3, jb3, out_ref):
    ms = ms0_ref[...] + ms1_ref[...]
    t = nf_ref[...] @ aw1n[...] + h_ref[...] @ aw1h[...] + ab1[...]
    r1 = jnp.tanh(jnp.maximum(t, 0.0) @ aw2[...] + ab2[...])
    r2 = _mlp_tanh_in(ms, bw1[...], bb1[...], bw2[...], bb2[...])
    h = jnp.concatenate([r1, r2], axis=1)
    h = h / jnp.sqrt(jnp.sum(h * h, axis=1, keepdims=True))
    z = jnp.maximum(h @ jw1[...] + jb1[...], 0.0)
    z = jnp.maximum(z @ jw2[...] + jb2[...], 0.0)
    out_ref[...] = z @ jw3[...] + jb3[...]


def _tc_node1(nf, h, mss, p, pj):
    grid = NP // BN
    ws = (p["a_w1"][0:16], p["a_w1"][16:32], p["a_b1"], p["a_w2"], p["a_b2"],
          p["b_w1"], p["b_b1"], p["b_w2"], p["b_b2"],
          pj["w1"], pj["b1"], pj["w2"], pj["b2"], pj["w3"], pj["b3"])
    return pl.pallas_call(
        _tc_node1_body,
        grid=(grid,),
        in_specs=[pl.BlockSpec((BN, 16), lambda i: (i, 0))
                  for _ in range(4)]
                 + [pl.BlockSpec(w.shape, lambda i: (0, 0)) for w in ws],
        out_specs=pl.BlockSpec((BN, 3), lambda i: (i, 0)),
        out_shape=jax.ShapeDtypeStruct((NP, 3), jnp.float32),
    )(nf, h, *mss, *ws)


def _pad_rows(x, rows):
    return jnp.pad(x, ((0, rows - x.shape[0]),) + ((0, 0),) * (x.ndim - 1))


def _as2d(b):
    return b.reshape(1, -1)


def _bd(w):
    a, b = w.shape
    out = jnp.zeros((8 * a, 8 * b), w.dtype)
    for j in range(8):
        out = out.at[j * a:(j + 1) * a, j * b:(j + 1) * b].set(w)
    return out


def kernel(node_features, flavour_indices, edge_index, edge_feat, params):
    nfeat = _pad_rows(node_features.astype(jnp.float32), NP)
    flav = _pad_rows(flavour_indices.astype(jnp.int32), NP)
    flav3 = flav.reshape(NP // BN, 1, BN)
    src = edge_index[0].astype(jnp.int32)
    dst = edge_index[1].astype(jnp.int32)
    src = jnp.pad(src, (0, EP - E))
    dst = jnp.pad(dst, (0, EP - E), constant_values=NP - 1)
    NCR = EPH // CH
    src2 = src.reshape(EP // CH, CH)
    dst2 = dst.reshape(EP // CH, CH)
    src2h = [src2[0:NCR], src2[NCR:]]
    dst2h = [dst2[0:NCR], dst2[NCR:]]
    ef = jnp.pad(edge_feat.astype(jnp.float32), (0, EP - E))
    efp = ef.reshape(EP8, 8)
    efph = [efp[0:EPH8], efp[EPH8:]]

    p = params
    nf = _tc_prep(nfeat, flav3, p["emb"])

    e0 = p["e0"]
    w1 = e0["w1"]
    w0s = ([_bd(w1[0:16]), _bd(w1[16:32])], _bd(w1[32:33]),
           _as2d(jnp.tile(e0["b1"], 8)), _bd(e0["w2"]),
           _as2d(jnp.tile(e0["b2"], 8)))
    gnf = []
    mp = None
    for hf in range(2):
        gdp, gsp = _sc_gather([(nf, dst2h[hf]), (nf, src2h[hf])])
        gnf.append((gdp, gsp))
        eh = _tc_edge([gdp, gsp], efph[hf], *w0s)
        mp = _sc_scatter(eh, dst2h[hf], init=mp)
    n0 = {k: (v if v.ndim == 2 else _as2d(v)) for k, v in p["n0"].items()}
    h = _tc_node0(nf, [mp[0], mp[1]], n0)

    e1 = p["e1"]
    w1 = e1["w1"]
    w1s = ([_bd(w1[0:16]), _bd(w1[16:32]), _bd(w1[32:48]), _bd(w1[48:64])],
           _bd(w1[64:65]),
           _as2d(jnp.tile(e1["b1"], 8)), _bd(e1["w2"]),
           _as2d(jnp.tile(e1["b2"], 8)))
    mp = None
    for hf in range(2):
        ghd, ghs = _sc_gather([(h, dst2h[hf]), (h, src2h[hf])])
        gdp, gsp = gnf[hf]
        eh = _tc_edge([gdp, ghd, gsp, ghs], efph[hf], *w1s)
        mp = _sc_scatter(eh, dst2h[hf], init=mp)
    n1 = {k: (v if v.ndim == 2 else _as2d(v)) for k, v in p["n1"].items()}
    pj = {k: (v if v.ndim == 2 else _as2d(v)) for k, v in p["jet"].items()}
    out = _tc_node1(nf, h, [mp[0], mp[1]], n1, pj)
    return out[:N]
